# baseline (speedup 1.0000x reference)
import jax
import jax.numpy as jnp
from jax import lax
from jax.experimental import pallas as pl
from jax.experimental.pallas import tpu as pltpu
from jax.experimental.pallas import tpu_sc as plsc

THRESH = 0.5

N_ROWS = 1048576
N_COLS = 64
NUM_CORES = 2
NUM_SUBCORES = 16
NUM_WORKERS = NUM_CORES * NUM_SUBCORES
ROWS_PER_WORKER = N_ROWS // NUM_WORKERS
LANES = 16
HALF = ROWS_PER_WORKER // 2
HALF_CHUNKS = HALF // LANES


def _body(x_hbm, idx_hbm, out_hbm, idx_v, col_v, out_v, sem):
    wid = lax.axis_index("s") * NUM_CORES + lax.axis_index("c")
    base = wid * ROWS_PER_WORKER

    iota = lax.iota(jnp.int32, LANES)
    even = 2 * iota
    odd = even + 1
    onef = jnp.ones((LANES,), jnp.float32)
    zerof = jnp.zeros((LANES,), jnp.float32)

    for h in range(2):
        pltpu.sync_copy(idx_hbm.at[pl.ds(base + h * HALF, HALF)], idx_v)
        pltpu.async_copy(x_hbm.at[idx_v], col_v, sem).wait()

        out_base = 2 * h * HALF

        def step(j, carry):
            v = col_v[pl.ds(j * LANES, LANES)]
            ge = jnp.where(v >= THRESH, onef, zerof)
            base2 = out_base + j * (2 * LANES)
            plsc.store_scatter(out_v, [base2 + even], ge)
            plsc.store_scatter(out_v, [base2 + odd], onef - ge)
            return carry

        lax.fori_loop(0, HALF_CHUNKS, step, 0, unroll=8)

    pltpu.sync_copy(out_v, out_hbm.at[pl.ds(2 * base, 2 * ROWS_PER_WORKER)])


@jax.jit
def _run(x):
    xf = x.reshape(-1)
    idx = jnp.arange(N_COLS - 1, N_ROWS * N_COLS, N_COLS, dtype=jnp.int32)
    mesh = plsc.VectorSubcoreMesh(core_axis_name="c", subcore_axis_name="s")
    flat = pl.kernel(
        _body,
        out_type=jax.ShapeDtypeStruct((2 * N_ROWS,), jnp.float32),
        mesh=mesh,
        scratch_types=[
            pltpu.VMEM((HALF,), jnp.int32),
            pltpu.VMEM((HALF,), jnp.float32),
            pltpu.VMEM((2 * ROWS_PER_WORKER,), jnp.float32),
            pltpu.SemaphoreType.DMA,
        ],
        compiler_params=pltpu.CompilerParams(needs_layout_passes=False),
    )(xf, idx)
    return flat.reshape(N_ROWS, 2)


def kernel(x):
    return _run(x)


# speedup vs baseline: 20.3863x; 20.3863x over previous
import jax
import jax.numpy as jnp
from jax import lax
from jax.experimental import pallas as pl
from jax.experimental.pallas import tpu as pltpu
from jax.experimental.pallas import tpu_sc as plsc

THRESH = 0.5

N_ROWS = 1048576
N_COLS = 64
NUM_CORES = 2
NUM_SUBCORES = 16
NUM_WORKERS = NUM_CORES * NUM_SUBCORES
COLS_PER_WORKER = N_ROWS // NUM_WORKERS
LANES = 16
CHUNK = 2048
N_CHUNKS = COLS_PER_WORKER // CHUNK
STEPS = CHUNK // LANES


def _body(xt_hbm, ge_hbm, lt_hbm, stripe_v, ge_v, lt_v):
    wid = lax.axis_index("s") * NUM_CORES + lax.axis_index("c")
    base = wid * COLS_PER_WORKER

    onef = jnp.ones((LANES,), jnp.float32)
    zerof = jnp.zeros((LANES,), jnp.float32)

    for c in range(N_CHUNKS):
        pltpu.sync_copy(
            xt_hbm.at[pl.ds(N_COLS - 8, 8), pl.ds(base + c * CHUNK, CHUNK)],
            stripe_v,
        )
        out_off = c * CHUNK

        def step(k, carry):
            v = stripe_v[7, pl.ds(k * LANES, LANES)]
            ge = jnp.where(v >= THRESH, onef, zerof)
            ge_v[pl.ds(out_off + k * LANES, LANES)] = ge
            lt_v[pl.ds(out_off + k * LANES, LANES)] = onef - ge
            return carry

        lax.fori_loop(0, STEPS, step, 0, unroll=8)

    pltpu.sync_copy(ge_v, ge_hbm.at[pl.ds(base, COLS_PER_WORKER)])
    pltpu.sync_copy(lt_v, lt_hbm.at[pl.ds(base, COLS_PER_WORKER)])


@jax.jit
def _run(x):
    xt = x.T
    mesh = plsc.VectorSubcoreMesh(core_axis_name="c", subcore_axis_name="s")
    ge, lt = pl.kernel(
        _body,
        out_type=(
            jax.ShapeDtypeStruct((N_ROWS,), jnp.float32),
            jax.ShapeDtypeStruct((N_ROWS,), jnp.float32),
        ),
        mesh=mesh,
        scratch_types=[
            pltpu.VMEM((8, CHUNK), jnp.float32),
            pltpu.VMEM((COLS_PER_WORKER,), jnp.float32),
            pltpu.VMEM((COLS_PER_WORKER,), jnp.float32),
        ],
    )(xt)
    return jnp.stack([ge, lt], axis=1)


def kernel(x):
    return _run(x)


# speedup vs baseline: 38.7252x; 1.8996x over previous
import jax
import jax.numpy as jnp
from jax import lax
from jax.experimental import pallas as pl
from jax.experimental.pallas import tpu as pltpu
from jax.experimental.pallas import tpu_sc as plsc

THRESH = 0.5

N_ROWS = 1048576
N_COLS = 64
NUM_CORES = 2
NUM_SUBCORES = 16
NUM_WORKERS = NUM_CORES * NUM_SUBCORES
LANES = 16

N_RUNS_TOTAL = (N_ROWS * N_COLS) // 128
RUNS_PER_WORKER = (N_ROWS // 128) // NUM_WORKERS
RUN_BASE = 7 * (N_ROWS // 128) * 8 + 7
OUT_WORDS_PER_WORKER = 2 * 128 * RUNS_PER_WORKER
STEPS = RUNS_PER_WORKER * 8


def _body(runs_hbm, out_hbm, idx_v, rows_v, out_v, sem):
    wid = lax.axis_index("s") * NUM_CORES + lax.axis_index("c")
    jbase = wid * RUNS_PER_WORKER

    iota = lax.iota(jnp.int32, LANES)
    onef = jnp.ones((LANES,), jnp.float32)
    zerof = jnp.zeros((LANES,), jnp.float32)

    for t in range(RUNS_PER_WORKER // LANES):
        idx_v[pl.ds(t * LANES, LANES)] = RUN_BASE + 8 * (jbase + t * LANES + iota)

    pltpu.async_copy(runs_hbm.at[idx_v], rows_v, sem).wait()

    def step(i, carry):
        t = i // 8
        m = i - t * 8
        v = rows_v[t, pl.ds(m * LANES, LANES)]
        ge = jnp.where(v >= THRESH, onef, zerof)
        off = t * 256 + m * LANES
        out_v[pl.ds(off, LANES)] = ge
        out_v[pl.ds(off + 128, LANES)] = onef - ge
        return carry

    lax.fori_loop(0, STEPS, step, 0, unroll=8)

    pltpu.sync_copy(
        out_v, out_hbm.at[pl.ds(wid * OUT_WORDS_PER_WORKER, OUT_WORDS_PER_WORKER)]
    )


@jax.jit
def _run(x):
    runs = (
        x.T.reshape(8, 8, N_ROWS // 128, 128)
        .transpose(0, 2, 1, 3)
        .reshape(N_RUNS_TOTAL, 128)
    )
    mesh = plsc.VectorSubcoreMesh(core_axis_name="c", subcore_axis_name="s")
    flat = pl.kernel(
        _body,
        out_type=jax.ShapeDtypeStruct((2 * N_ROWS,), jnp.float32),
        mesh=mesh,
        scratch_types=[
            pltpu.VMEM((RUNS_PER_WORKER,), jnp.int32),
            pltpu.VMEM((RUNS_PER_WORKER, 128), jnp.float32),
            pltpu.VMEM((OUT_WORDS_PER_WORKER,), jnp.float32),
            pltpu.SemaphoreType.DMA,
        ],
    )(runs)
    return flat.reshape(N_ROWS // 128, 2, 128).transpose(0, 2, 1).reshape(N_ROWS, 2)


def kernel(x):
    return _run(x)


# speedup vs baseline: 39.8342x; 1.0286x over previous
import jax
import jax.numpy as jnp
from jax import lax
from jax.experimental import pallas as pl
from jax.experimental.pallas import tpu as pltpu
from jax.experimental.pallas import tpu_sc as plsc

THRESH = 0.5

N_ROWS = 1048576
N_COLS = 64
NUM_CORES = 2
NUM_SUBCORES = 16
NUM_WORKERS = NUM_CORES * NUM_SUBCORES
LANES = 16

N_RUNS_TOTAL = (N_ROWS * N_COLS) // 128
RUNS_PER_WORKER = (N_ROWS // 128) // NUM_WORKERS
RUN_BASE = 7 * (N_ROWS // 128) * 8 + 7
OUT_WORDS_PER_WORKER = 2 * 128 * RUNS_PER_WORKER
STEPS = RUNS_PER_WORKER * 8


N_OUT_CHUNKS = 4
RUNS_PER_CHUNK = RUNS_PER_WORKER // N_OUT_CHUNKS
CHUNK_STEPS = RUNS_PER_CHUNK * 8
CHUNK_OUT_WORDS = OUT_WORDS_PER_WORKER // N_OUT_CHUNKS


def _body(runs_hbm, out_hbm, idx_v, rows_v, out_v, gsem, osem):
    wid = lax.axis_index("s") * NUM_CORES + lax.axis_index("c")
    jbase = wid * RUNS_PER_WORKER

    iota = lax.iota(jnp.int32, LANES)
    onef = jnp.ones((LANES,), jnp.float32)
    zerof = jnp.zeros((LANES,), jnp.float32)

    for t in range(RUNS_PER_WORKER // LANES):
        idx_v[pl.ds(t * LANES, LANES)] = RUN_BASE + 8 * (jbase + t * LANES + iota)

    gather = pltpu.async_copy(runs_hbm.at[idx_v], rows_v, gsem)
    gather.wait()

    out_base_hbm = wid * OUT_WORDS_PER_WORKER
    copies = []
    for c in range(N_OUT_CHUNKS):
        t0 = c * RUNS_PER_CHUNK

        def step(i, carry):
            t = t0 + i // 8
            m = i - (i // 8) * 8
            v = rows_v[t, pl.ds(m * LANES, LANES)]
            ge = jnp.where(v >= THRESH, onef, zerof)
            off = t * 256 + m * LANES
            out_v[pl.ds(off, LANES)] = ge
            out_v[pl.ds(off + 128, LANES)] = onef - ge
            return carry

        lax.fori_loop(0, CHUNK_STEPS, step, 0, unroll=8)
        copies.append(
            pltpu.async_copy(
                out_v.at[pl.ds(c * CHUNK_OUT_WORDS, CHUNK_OUT_WORDS)],
                out_hbm.at[pl.ds(out_base_hbm + c * CHUNK_OUT_WORDS, CHUNK_OUT_WORDS)],
                osem,
            )
        )
    for cp in copies:
        cp.wait()


@jax.jit
def _run(x):
    runs = (
        x.T.reshape(8, 8, N_ROWS // 128, 128)
        .transpose(0, 2, 1, 3)
        .reshape(N_RUNS_TOTAL, 128)
    )
    mesh = plsc.VectorSubcoreMesh(core_axis_name="c", subcore_axis_name="s")
    flat = pl.kernel(
        _body,
        out_type=jax.ShapeDtypeStruct((2 * N_ROWS,), jnp.float32),
        mesh=mesh,
        scratch_types=[
            pltpu.VMEM((RUNS_PER_WORKER,), jnp.int32),
            pltpu.VMEM((RUNS_PER_WORKER, 128), jnp.float32),
            pltpu.VMEM((OUT_WORDS_PER_WORKER,), jnp.float32),
            pltpu.SemaphoreType.DMA,
            pltpu.SemaphoreType.DMA,
        ],
    )(runs)
    return flat.reshape(N_ROWS // 128, 2, 128).transpose(0, 2, 1).reshape(N_ROWS, 2)


def kernel(x):
    return _run(x)


# speedup vs baseline: 39.9262x; 1.0023x over previous
import jax
import jax.numpy as jnp
from jax import lax
from jax.experimental import pallas as pl
from jax.experimental.pallas import tpu as pltpu
from jax.experimental.pallas import tpu_sc as plsc

THRESH = 0.5

N_ROWS = 1048576
N_COLS = 64
NUM_CORES = 2
NUM_SUBCORES = 16
NUM_WORKERS = NUM_CORES * NUM_SUBCORES
LANES = 16

N_RUNS_TOTAL = (N_ROWS * N_COLS) // 128
RUNS_PER_WORKER = (N_ROWS // 128) // NUM_WORKERS
RUN_BASE = 7 * (N_ROWS // 128) * 8 + 7
OUT_WORDS_PER_WORKER = 2 * 128 * RUNS_PER_WORKER
STEPS = RUNS_PER_WORKER * 8


N_OUT_CHUNKS = 4
RUNS_PER_CHUNK = RUNS_PER_WORKER // N_OUT_CHUNKS
CHUNK_STEPS = RUNS_PER_CHUNK * 8
CHUNK_OUT_WORDS = OUT_WORDS_PER_WORKER // N_OUT_CHUNKS


def _body(runs_hbm, out_hbm, idx_v, rows_v, out_v, gsem, osem):
    wid = lax.axis_index("s") * NUM_CORES + lax.axis_index("c")
    jbase = wid * RUNS_PER_WORKER

    iota = lax.iota(jnp.int32, LANES)
    onef = jnp.ones((LANES,), jnp.float32)
    zerof = jnp.zeros((LANES,), jnp.float32)

    for t in range(RUNS_PER_WORKER // LANES):
        idx_v[pl.ds(t * LANES, LANES)] = RUN_BASE + 8 * (jbase + t * LANES + iota)

    gather = pltpu.async_copy(runs_hbm.at[idx_v], rows_v, gsem)
    gather.wait()

    out_base_hbm = wid * OUT_WORDS_PER_WORKER
    copies = []
    for c in range(N_OUT_CHUNKS):
        t0 = c * RUNS_PER_CHUNK

        def step(i, carry):
            t = t0 + i // 8
            m = i - (i // 8) * 8
            v = rows_v[t, pl.ds(m * LANES, LANES)]
            ge = jnp.where(v >= THRESH, onef, zerof)
            off = t * 256 + m * LANES
            out_v[pl.ds(off, LANES)] = ge
            out_v[pl.ds(off + 128, LANES)] = onef - ge
            return carry

        lax.fori_loop(0, CHUNK_STEPS, step, 0, unroll=8)
        copies.append(
            pltpu.async_copy(
                out_v.at[pl.ds(c * CHUNK_OUT_WORDS, CHUNK_OUT_WORDS)],
                out_hbm.at[pl.ds(out_base_hbm + c * CHUNK_OUT_WORDS, CHUNK_OUT_WORDS)],
                osem,
            )
        )
    for cp in copies:
        cp.wait()


@jax.jit
def _run(x):
    runs = (
        x.T.reshape(8, 8, N_ROWS // 128, 128)
        .transpose(0, 2, 1, 3)
        .reshape(N_RUNS_TOTAL, 128)
    )
    mesh = plsc.VectorSubcoreMesh(core_axis_name="c", subcore_axis_name="s")
    flat = pl.kernel(
        _body,
        out_type=jax.ShapeDtypeStruct((2 * N_ROWS,), jnp.float32),
        mesh=mesh,
        scratch_types=[
            pltpu.VMEM((RUNS_PER_WORKER,), jnp.int32),
            pltpu.VMEM((RUNS_PER_WORKER, 128), jnp.float32),
            pltpu.VMEM((OUT_WORDS_PER_WORKER,), jnp.float32),
            pltpu.SemaphoreType.DMA,
            pltpu.SemaphoreType.DMA,
        ],
        compiler_params=pltpu.CompilerParams(skip_device_barrier=True),
    )(runs)
    return flat.reshape(N_ROWS // 128, 2, 128).transpose(0, 2, 1).reshape(N_ROWS, 2)


def kernel(x):
    return _run(x)
